# baseline (speedup 1.0000x reference)
import functools

import jax
import jax.numpy as jnp
from jax import lax
from jax.experimental import pallas as pl
from jax.experimental.pallas import tpu as pltpu
from jax.experimental.pallas import tpu_sc as plsc

_B = 16384
_D = 16
_DP = _D // 2
_V = 1000000
_NW = 32
_BPW = _B // _NW
_CHUNK = 128
_NCH = _BPW // _CHUNK
_GROUPS = _BPW // 16

_TC_T = 512
_TC_STEPS = -(-_V // (128 * _TC_T))
_YROWS = _TC_STEPS * _TC_T * _DP


def _detile_body(src, dst):
    for s in range(_TC_T):
        sl = slice(s * 128, (s + 1) * 128)
        lo = lax.bitcast_convert_type(
            src[0:_DP, sl].astype(jnp.bfloat16), jnp.uint16
        ).astype(jnp.uint32)
        hi = lax.bitcast_convert_type(
            src[_DP:_D, sl].astype(jnp.bfloat16), jnp.uint16
        ).astype(jnp.uint32)
        dst[s * _DP:(s + 1) * _DP, :] = lo | (hi << 16)


_detile1 = pl.pallas_call(
    _detile_body,
    grid=(_TC_STEPS,),
    in_specs=[pl.BlockSpec((_D, 128 * _TC_T), lambda t: (0, t))],
    out_specs=pl.BlockSpec((_TC_T * _DP, 128), lambda t: (t, 0)),
    out_shape=jax.ShapeDtypeStruct((_YROWS, 128), jnp.uint32),
)

_mesh = plsc.VectorSubcoreMesh(core_axis_name="c", subcore_axis_name="s")
_SC_PARAMS = pltpu.CompilerParams(use_tc_tiling_on_sc=False,
                                  needs_layout_passes=False)


def _worker_id():
    return lax.axis_index("s") * 2 + lax.axis_index("c")


def _stage_offsets(idx_hbm, base, idx_v, w_ref):
    pltpu.sync_copy(idx_hbm.at[pl.ds(base, _BPW)], idx_v)

    def word_offsets(k, carry):
        iv = idx_v[pl.ds(k * 16, 16)]
        w0 = ((iv >> 7) << 10) + (iv & 127)
        for dp in range(_DP):
            w_ref[dp, pl.ds(k * 16, 16)] = w0 + dp * 128
        return carry

    lax.fori_loop(0, _BPW // 16, word_offsets, 0)


def _fire_gathers(tab, w_ref, r_ref, sem):
    flat = tab.at[0]
    copies = []
    for dp in range(_DP):
        for c in range(_NCH):
            sl = pl.ds(c * _CHUNK, _CHUNK)
            copies.append(
                pltpu.async_copy(flat.at[w_ref.at[dp, sl]], r_ref.at[dp, sl], sem))
    return copies


@functools.partial(
    pl.kernel,
    out_type=jax.ShapeDtypeStruct((_NW, _DP, _BPW), jnp.uint32),
    mesh=_mesh,
    scratch_types=[
        pltpu.VMEM((_BPW,), jnp.int32),
        pltpu.VMEM((_DP, _BPW), jnp.int32),
        pltpu.VMEM((_DP, _BPW), jnp.uint32),
        pltpu.SemaphoreType.DMA,
    ],
    compiler_params=_SC_PARAMS,
)
def _gather1(list_idx, yl, gw, idx_v, w_v, r_v, sem):
    wid = _worker_id()
    _stage_offsets(list_idx, wid * _BPW, idx_v, w_v)
    for cp in _fire_gathers(yl, w_v, r_v, sem):
        cp.wait()
    pltpu.sync_copy(r_v, gw.at[wid])


@functools.partial(
    pl.kernel,
    out_type=jax.ShapeDtypeStruct((_B,), jnp.float32),
    mesh=_mesh,
    scratch_types=[
        pltpu.VMEM((_BPW,), jnp.int32),
        pltpu.VMEM((_DP, _BPW), jnp.int32),
        pltpu.VMEM((_DP, _BPW), jnp.uint32),
        pltpu.VMEM((_DP, _BPW), jnp.uint32),
        pltpu.VMEM((_BPW,), jnp.float32),
        pltpu.SemaphoreType.DMA,
    ],
    compiler_params=_SC_PARAMS,
)
def _final(item_idx, yi, gw, out, idx_v, w_v, ri_v, rl_v, out_v, sem):
    wid = _worker_id()
    base = wid * _BPW
    _stage_offsets(item_idx, base, idx_v, w_v)
    copies = _fire_gathers(yi, w_v, ri_v, sem)
    pltpu.sync_copy(gw.at[wid], rl_v)
    for cp in copies:
        cp.wait()

    mask_hi = jnp.uint32(0xFFFF0000)

    def group(g, carry):
        sl = pl.ds(g * 16, 16)
        acc = jnp.zeros((16,), jnp.float32)
        for dp in range(_DP):
            wl = rl_v[dp, sl]
            wi = ri_v[dp, sl]
            l_lo = plsc.bitcast(wl << 16, jnp.float32)
            i_lo = plsc.bitcast(wi << 16, jnp.float32)
            l_hi = plsc.bitcast(wl & mask_hi, jnp.float32)
            i_hi = plsc.bitcast(wi & mask_hi, jnp.float32)
            acc = acc + l_lo * i_lo + l_hi * i_hi
        out_v[sl] = 1.0 / (1.0 + jnp.exp(-acc))
        return carry

    lax.fori_loop(0, _GROUPS, group, 0)
    pltpu.sync_copy(out_v, out.at[pl.ds(base, _BPW)])


def kernel(user_indices, list_indices, item_indices,
           user_table, list_table, item_table):
    del user_indices, user_table
    yl = _detile1(list_table.T)
    gl = _gather1(list_indices.astype(jnp.int32), yl)
    yi = _detile1(item_table.T)
    return _final(item_indices.astype(jnp.int32), yi, gl)


# speedup vs baseline: 1.0285x; 1.0285x over previous
import functools

import jax
import jax.numpy as jnp
from jax import lax
from jax.experimental import pallas as pl
from jax.experimental.pallas import tpu as pltpu
from jax.experimental.pallas import tpu_sc as plsc

_B = 16384
_D = 16
_DP = _D // 2
_V = 1000000
_NW = 32
_BPW = _B // _NW
_CHUNK = 128
_NCH = _BPW // _CHUNK
_GROUPS = _BPW // 16

_TC_T = 1024
_TC_STEPS = -(-_V // (128 * _TC_T))
_YROWS = _TC_STEPS * _TC_T * _DP


def _detile_body(ltab_ref, itab_ref, yl_ref, yi_ref):
    for src, dst in ((ltab_ref, yl_ref), (itab_ref, yi_ref)):
        for s in range(_TC_T):
            sl = slice(s * 128, (s + 1) * 128)
            lo = lax.bitcast_convert_type(
                src[0:_DP, sl].astype(jnp.bfloat16), jnp.uint16
            ).astype(jnp.uint32)
            hi = lax.bitcast_convert_type(
                src[_DP:_D, sl].astype(jnp.bfloat16), jnp.uint16
            ).astype(jnp.uint32)
            dst[s * _DP:(s + 1) * _DP, :] = lo | (hi << 16)


_detile = pl.pallas_call(
    _detile_body,
    grid=(_TC_STEPS,),
    in_specs=[
        pl.BlockSpec((_D, 128 * _TC_T), lambda t: (0, t)),
        pl.BlockSpec((_D, 128 * _TC_T), lambda t: (0, t)),
    ],
    out_specs=[
        pl.BlockSpec((_TC_T * _DP, 128), lambda t: (t, 0)),
        pl.BlockSpec((_TC_T * _DP, 128), lambda t: (t, 0)),
    ],
    out_shape=[
        jax.ShapeDtypeStruct((_YROWS, 128), jnp.uint32),
        jax.ShapeDtypeStruct((_YROWS, 128), jnp.uint32),
    ],
)

_mesh = plsc.VectorSubcoreMesh(core_axis_name="c", subcore_axis_name="s")


@functools.partial(
    pl.kernel,
    out_type=jax.ShapeDtypeStruct((_B,), jnp.float32),
    mesh=_mesh,
    scratch_types=[
        pltpu.VMEM((_BPW,), jnp.int32),
        pltpu.VMEM((_BPW,), jnp.int32),
        pltpu.VMEM((_DP, _BPW), jnp.int32),
        pltpu.VMEM((_DP, _BPW), jnp.int32),
        pltpu.VMEM((_DP, _BPW), jnp.uint32),
        pltpu.VMEM((_DP, _BPW), jnp.uint32),
        pltpu.VMEM((_BPW,), jnp.float32),
        pltpu.SemaphoreType.DMA,
    ],
    compiler_params=pltpu.CompilerParams(use_tc_tiling_on_sc=False,
                                         needs_layout_passes=False),
)
def _mf_sc(list_idx, item_idx, yl, yi, out,
           idxl_v, idxi_v, wl_v, wi_v, rl_v, ri_v, out_v, sem):
    wid = lax.axis_index("s") * 2 + lax.axis_index("c")
    base = wid * _BPW

    pltpu.sync_copy(list_idx.at[pl.ds(base, _BPW)], idxl_v)
    pltpu.sync_copy(item_idx.at[pl.ds(base, _BPW)], idxi_v)

    def word_offsets(k, carry):
        for idx_ref, w_ref in ((idxl_v, wl_v), (idxi_v, wi_v)):
            iv = idx_ref[pl.ds(k * 16, 16)]
            w0 = ((iv >> 7) << 10) + (iv & 127)
            for dp in range(_DP):
                w_ref[dp, pl.ds(k * 16, 16)] = w0 + dp * 128
        return carry

    lax.fori_loop(0, _BPW // 16, word_offsets, 0)

    copies = []
    for tab, w_ref, r_ref in ((yl, wl_v, rl_v), (yi, wi_v, ri_v)):
        flat = tab.at[0]
        for dp in range(_DP):
            for c in range(_NCH):
                sl = pl.ds(c * _CHUNK, _CHUNK)
                copies.append(
                    pltpu.async_copy(flat.at[w_ref.at[dp, sl]], r_ref.at[dp, sl], sem))
    for cp in copies:
        cp.wait()

    mask_hi = jnp.uint32(0xFFFF0000)

    def group(g, carry):
        sl = pl.ds(g * 16, 16)
        acc = jnp.zeros((16,), jnp.float32)
        for dp in range(_DP):
            wl = rl_v[dp, sl]
            wi = ri_v[dp, sl]
            l_lo = plsc.bitcast(wl << 16, jnp.float32)
            i_lo = plsc.bitcast(wi << 16, jnp.float32)
            l_hi = plsc.bitcast(wl & mask_hi, jnp.float32)
            i_hi = plsc.bitcast(wi & mask_hi, jnp.float32)
            acc = acc + l_lo * i_lo + l_hi * i_hi
        out_v[sl] = 1.0 / (1.0 + jnp.exp(-acc))
        return carry

    lax.fori_loop(0, _GROUPS, group, 0)
    pltpu.sync_copy(out_v, out.at[pl.ds(base, _BPW)])


def kernel(user_indices, list_indices, item_indices,
           user_table, list_table, item_table):
    del user_indices, user_table
    yl, yi = _detile(list_table.T, item_table.T)
    return _mf_sc(list_indices.astype(jnp.int32),
                  item_indices.astype(jnp.int32),
                  yl, yi)
